# baseline (speedup 1.0000x reference)
import jax
import jax.numpy as jnp
from jax import lax
from jax.experimental import pallas as pl
from jax.experimental.pallas import tpu as pltpu
from jax.experimental.pallas import tpu_sc as plsc

B, S, T, D = 1024, 200, 8, 64
BS = B * S
NC, NS = 2, 16
NW = NC * NS
TPW = BS // NW
C = 128
NCHUNK = TPW // C
NF = 11
KW = 12


def _dense_chunk(wcat_f, feat_f, acc_v):
    iota = lax.iota(jnp.int32, 16)
    ones = jnp.full((16,), 1, jnp.int32)
    for h in range(2):
        offs = [h * 32 + jj * 16 for jj in range(2)]
        wv = [[plsc.load_gather(wcat_f, [iota + (k * D + offs[jj])])
               for jj in range(2)] for k in range(NF)]
        bv = [plsc.load_gather(wcat_f, [iota + (NF * D + offs[jj])])
              for jj in range(2)]

        def tok(i, ci):
            fb = [plsc.load_gather(feat_f, [ci + k * C]) for k in range(NF)]
            for jj in range(2):
                acc = bv[jj]
                for k in range(NF):
                    acc = acc + fb[k] * wv[k][jj]
                acc_v[i, pl.ds(offs[jj], 16)] = acc
            return ci + ones

        lax.fori_loop(0, C, tok, jnp.zeros((16,), jnp.int32))


def _sc_body(feat_hbm, wcat_hbm, sp_ids, ev_ids, bu_ids, eq_ids, po_ids,
             sp_tab, ev_tab, bu_tab, eq_tab, po_tab,
             out_hbm,
             wcat_f, feat_f, idx_v, acc_v, sem_in, sem_g):
    wid = lax.axis_index("s") * NC + lax.axis_index("c")
    wbase = wid * TPW
    ids_all = (sp_ids, ev_ids, bu_ids, eq_ids, po_ids)
    tabs = (sp_tab, ev_tab, bu_tab, eq_tab, po_tab)

    pltpu.sync_copy(wcat_hbm, wcat_f.at[pl.ds(0, KW * D)])

    def chunk(g, _):
        tok0 = wbase + g * C
        d_in = [pltpu.async_copy(ids.at[pl.ds(tok0, C)], idx_v.at[t], sem_in)
                for t, ids in enumerate(ids_all)]
        for k in range(NF):
            d_in.append(pltpu.async_copy(
                feat_hbm.at[k, pl.ds(tok0, C)], feat_f.at[pl.ds(k * C, C)],
                sem_in))
        for dsc in d_in:
            dsc.wait()

        _dense_chunk(wcat_f, feat_f, acc_v)

        d_g = [pltpu.async_copy(tab.at[idx_v.at[t]], acc_v, sem_g, add=True)
               for t, tab in enumerate(tabs)]
        for dsc in d_g:
            dsc.wait()

        pltpu.sync_copy(acc_v, out_hbm.at[pl.ds(tok0, C)])
        return 0

    lax.fori_loop(0, NCHUNK, chunk, 0)


def _sc_run(feat, wcat, sp_i, ev_i, bu_i, eq_i, po_i,
            sp_t, ev_t, bu_t, eq_t, po_t):
    mesh = plsc.VectorSubcoreMesh(core_axis_name="c", subcore_axis_name="s",
                                  num_cores=NC, num_subcores=NS)
    kern = pl.kernel(
        _sc_body,
        out_type=jax.ShapeDtypeStruct((BS, D), jnp.float32),
        mesh=mesh,
        scratch_types=[
            pltpu.VMEM((KW * D,), jnp.float32),
            pltpu.VMEM((NF * C,), jnp.float32),
            pltpu.VMEM((5, C), jnp.int32),
            pltpu.VMEM((C, D), jnp.float32),
            pltpu.SemaphoreType.DMA,
            pltpu.SemaphoreType.DMA,
        ],
        compiler_params=pltpu.CompilerParams(use_tc_tiling_on_sc=False,
                                            needs_layout_passes=False),
    )
    return kern(feat, wcat.reshape(KW * D), sp_i, ev_i, bu_i, eq_i, po_i,
                sp_t, ev_t, bu_t, eq_t, po_t)


def kernel(time_features, spatial_ids, population, event_counts, event_types,
           building_counts, building_type_ids, equipment_ids, positions,
           W_time, b_time, spatial_table, W_pop, b_pop, W_ec, b_ec,
           event_type_table, W_bc, b_bc, building_type_table,
           equipment_table, position_table):
    f32 = jnp.float32
    feat = jnp.concatenate(
        [time_features.reshape(BS, T).astype(f32).T,
         population.reshape(1, BS).astype(f32),
         event_counts.reshape(1, BS).astype(f32),
         building_counts.reshape(1, BS).astype(f32)], axis=0)
    bias = (b_time + b_pop + b_ec + b_bc).astype(f32)
    wcat = jnp.concatenate(
        [W_time.astype(f32), W_pop.astype(f32), W_ec.astype(f32),
         W_bc.astype(f32), bias[None, :]], axis=0)

    out = _sc_run(
        feat, wcat,
        spatial_ids.reshape(BS).astype(jnp.int32),
        event_types.reshape(BS).astype(jnp.int32),
        building_type_ids.reshape(BS).astype(jnp.int32),
        equipment_ids.reshape(BS).astype(jnp.int32),
        positions.reshape(BS).astype(jnp.int32),
        spatial_table.astype(f32), event_type_table.astype(f32),
        building_type_table.astype(f32), equipment_table.astype(f32),
        position_table.astype(f32))
    return out.reshape(B, S, D)


# speedup vs baseline: 1.2216x; 1.2216x over previous
import jax
import jax.numpy as jnp
from jax import lax
from jax.experimental import pallas as pl
from jax.experimental.pallas import tpu as pltpu
from jax.experimental.pallas import tpu_sc as plsc

B, S, T, D = 1024, 200, 8, 64
BS = B * S
NC, NS = 2, 16
NW = NC * NS
TPW = BS // NW
C = 128
NCHUNK = TPW // C
NF = 11
KW = 12


def _dense_chunk(wcat_f, feat_f, acc_v):
    iota = lax.iota(jnp.int32, 16)
    ones = jnp.full((16,), 1, jnp.int32)
    for h in range(2):
        offs = [h * 32 + jj * 16 for jj in range(2)]
        wv = [[plsc.load_gather(wcat_f, [iota + (k * D + offs[jj])])
               for jj in range(2)] for k in range(NF)]
        bv = [plsc.load_gather(wcat_f, [iota + (NF * D + offs[jj])])
              for jj in range(2)]

        def tok(i, ci):
            fb = [plsc.load_gather(feat_f, [ci + k * C]) for k in range(NF)]
            for jj in range(2):
                acc = bv[jj]
                for k in range(NF):
                    acc = acc + fb[k] * wv[k][jj]
                acc_v[i, pl.ds(offs[jj], 16)] = acc
            return ci + ones

        lax.fori_loop(0, C, tok, jnp.zeros((16,), jnp.int32), unroll=4)


def _sc_body(feat_hbm, wcat_hbm, sp_ids, ev_ids, bu_ids, eq_ids, po_ids,
             sp_tab, ev_tab, bu_tab, eq_tab, po_tab,
             out_hbm,
             wcat_f, feat_f0, feat_f1, idx_v0, idx_v1, acc_v0, acc_v1,
             sem_in0, sem_in1, sem_g0, sem_g1, sem_out0, sem_out1):
    wid = lax.axis_index("s") * NC + lax.axis_index("c")
    wbase = wid * TPW
    ids_all = (sp_ids, ev_ids, bu_ids, eq_ids, po_ids)
    tabs = (sp_tab, ev_tab, bu_tab, eq_tab, po_tab)
    feat_f = (feat_f0, feat_f1)
    idx_v = (idx_v0, idx_v1)
    acc_v = (acc_v0, acc_v1)
    sem_in = (sem_in0, sem_in1)
    sem_g = (sem_g0, sem_g1)
    sem_out = (sem_out0, sem_out1)

    pltpu.sync_copy(wcat_hbm, wcat_f.at[pl.ds(0, KW * D)])

    def fire_in(g, p):
        tok0 = wbase + g * C
        for t, ids in enumerate(ids_all):
            pltpu.async_copy(ids.at[pl.ds(tok0, C)], idx_v[p].at[t], sem_in[p])
        for k in range(NF):
            pltpu.async_copy(feat_hbm.at[k, pl.ds(tok0, C)],
                             feat_f[p].at[pl.ds(k * C, C)], sem_in[p])

    def drain_in(p):
        for t in range(5):
            pltpu.make_async_copy(ids_all[t].at[pl.ds(0, C)],
                                  idx_v[p].at[t], sem_in[p]).wait()
        for k in range(NF):
            pltpu.make_async_copy(feat_hbm.at[k, pl.ds(0, C)],
                                  feat_f[p].at[pl.ds(k * C, C)],
                                  sem_in[p]).wait()

    def fire_gathers(g, p):
        for t, tab in enumerate(tabs):
            pltpu.async_copy(tab.at[idx_v[p].at[t]], acc_v[p], sem_g[p],
                             add=True)

    def drain_gathers(p):
        for _ in range(5):
            pltpu.make_async_copy(out_hbm.at[pl.ds(0, C)], acc_v[p],
                                  sem_g[p]).wait()

    def fire_out(g, p):
        tok0 = wbase + g * C
        pltpu.async_copy(acc_v[p], out_hbm.at[pl.ds(tok0, C)], sem_out[p])

    def drain_out(p):
        pltpu.make_async_copy(acc_v[p], out_hbm.at[pl.ds(0, C)],
                              sem_out[p]).wait()

    def dense(p):
        _dense_chunk(wcat_f, feat_f[p], acc_v[p])

    fire_in(0, 0)
    fire_in(1, 1)
    drain_in(0)
    dense(0)
    fire_gathers(0, 0)
    drain_in(1)
    dense(1)
    drain_gathers(0)
    fire_out(0, 0)
    fire_in(2, 0)
    fire_gathers(1, 1)

    def two(gg, _):
        for p in range(2):
            g = 2 * gg + p
            q = 1 - p
            drain_in(p)
            drain_out(p)
            dense(p)
            drain_gathers(q)
            fire_out(g - 1, q)
            gn = jnp.minimum(g + 1, NCHUNK - 1)
            fire_in(gn, q)
            fire_gathers(g, p)
        return 0

    lax.fori_loop(1, NCHUNK // 2, two, 0)

    drain_gathers(1)
    fire_out(NCHUNK - 1, 1)
    drain_in(0)
    drain_out(0)
    drain_out(1)


def _sc_run(feat, wcat, sp_i, ev_i, bu_i, eq_i, po_i,
            sp_t, ev_t, bu_t, eq_t, po_t):
    mesh = plsc.VectorSubcoreMesh(core_axis_name="c", subcore_axis_name="s",
                                  num_cores=NC, num_subcores=NS)
    kern = pl.kernel(
        _sc_body,
        out_type=jax.ShapeDtypeStruct((BS, D), jnp.float32),
        mesh=mesh,
        scratch_types=[
            pltpu.VMEM((KW * D,), jnp.float32),
            pltpu.VMEM((NF * C,), jnp.float32),
            pltpu.VMEM((NF * C,), jnp.float32),
            pltpu.VMEM((5, C), jnp.int32),
            pltpu.VMEM((5, C), jnp.int32),
            pltpu.VMEM((C, D), jnp.float32),
            pltpu.VMEM((C, D), jnp.float32),
            pltpu.SemaphoreType.DMA,
            pltpu.SemaphoreType.DMA,
            pltpu.SemaphoreType.DMA,
            pltpu.SemaphoreType.DMA,
            pltpu.SemaphoreType.DMA,
            pltpu.SemaphoreType.DMA,
        ],
        compiler_params=pltpu.CompilerParams(use_tc_tiling_on_sc=False,
                                            needs_layout_passes=False),
    )
    return kern(feat, wcat.reshape(KW * D), sp_i, ev_i, bu_i, eq_i, po_i,
                sp_t, ev_t, bu_t, eq_t, po_t)


def kernel(time_features, spatial_ids, population, event_counts, event_types,
           building_counts, building_type_ids, equipment_ids, positions,
           W_time, b_time, spatial_table, W_pop, b_pop, W_ec, b_ec,
           event_type_table, W_bc, b_bc, building_type_table,
           equipment_table, position_table):
    f32 = jnp.float32
    feat = jnp.concatenate(
        [time_features.reshape(BS, T).astype(f32).T,
         population.reshape(1, BS).astype(f32),
         event_counts.reshape(1, BS).astype(f32),
         building_counts.reshape(1, BS).astype(f32)], axis=0)
    bias = (b_time + b_pop + b_ec + b_bc).astype(f32)
    wcat = jnp.concatenate(
        [W_time.astype(f32), W_pop.astype(f32), W_ec.astype(f32),
         W_bc.astype(f32), bias[None, :]], axis=0)

    out = _sc_run(
        feat, wcat,
        spatial_ids.reshape(BS).astype(jnp.int32),
        event_types.reshape(BS).astype(jnp.int32),
        building_type_ids.reshape(BS).astype(jnp.int32),
        equipment_ids.reshape(BS).astype(jnp.int32),
        positions.reshape(BS).astype(jnp.int32),
        spatial_table.astype(f32), event_type_table.astype(f32),
        building_type_table.astype(f32), equipment_table.astype(f32),
        position_table.astype(f32))
    return out.reshape(B, S, D)
